# Initial kernel scaffold; baseline (speedup 1.0000x reference)
import jax
import jax.numpy as jnp
from jax.experimental import pallas as pl


def kernel(inputs_flatten, embed, cluster_size, embed_avg):
    raise NotImplementedError("write your pallas kernel here")


# speedup vs baseline: 7.3822x; 7.3822x over previous
import functools

import jax
import jax.numpy as jnp
from jax import lax
from jax.experimental import pallas as pl
from jax.experimental.pallas import tpu as pltpu
from jax.experimental.pallas import tpu_sc as plsc

N = 18432
D = 256
K = 8192
DECAY = 0.1
EPS = 1e-05


BN = 512
BK = 1024
NB = N // BN
KB = K // BK


def _e2_body(e_ref, out_ref):
    eb = e_ref[...].reshape(8, 128, D)
    out_ref[...] = jnp.sum(eb * eb, axis=2)


def _e2(e):
    return pl.pallas_call(
        _e2_body,
        grid=(K // 1024,),
        in_specs=[pl.BlockSpec((1024, D), lambda r: (r, 0))],
        out_specs=pl.BlockSpec((8, 128), lambda r: (r, 0)),
        out_shape=jax.ShapeDtypeStruct((K // 128, 128), jnp.float32),
    )(e)


def _argmin_body(x_ref, e_ref, e2_ref, idx_ref, min_s, arg_s, scores_s):
    k = pl.program_id(1)
    scores_s[...] = -2.0 * lax.dot_general(
        x_ref[...], e_ref[...], (((1,), (1,)), ((), ())),
        preferred_element_type=jnp.float32)

    @pl.when(k == 0)
    def _():
        min_s[...] = jnp.full((BN, 128), jnp.inf, jnp.float32)
        arg_s[...] = jnp.zeros((BN, 128), jnp.int32)

    lane = lax.broadcasted_iota(jnp.int32, (BN, 128), 1)
    for g in range(BK // 128):
        sub = scores_s[:, g * 128:(g + 1) * 128] + e2_ref[pl.ds(g, 1), :]
        col = lane + (k * BK + g * 128)
        better = sub < min_s[...]
        min_s[...] = jnp.where(better, sub, min_s[...])
        arg_s[...] = jnp.where(better, col, arg_s[...])

    @pl.when(k == KB - 1)
    def _():
        m = min_s[...]
        a = arg_s[...]
        best = jnp.min(m, axis=1, keepdims=True)
        idx_ref[...] = jnp.min(jnp.where(m == best, a, K), axis=1)


def _argmin(x, e, e2):
    return pl.pallas_call(
        _argmin_body,
        grid=(NB, KB),
        in_specs=[
            pl.BlockSpec((BN, D), lambda i, k: (i, 0)),
            pl.BlockSpec((BK, D), lambda i, k: (k, 0)),
            pl.BlockSpec((BK // 128, 128), lambda i, k: (k, 0)),
        ],
        out_specs=pl.BlockSpec((BN,), lambda i, k: (i,)),
        out_shape=jax.ShapeDtypeStruct((N,), jnp.int32),
        scratch_shapes=[
            pltpu.VMEM((BN, 128), jnp.float32),
            pltpu.VMEM((BN, 128), jnp.int32),
            pltpu.VMEM((BN, BK), jnp.float32),
        ],
    )(x, e, e2)


ROWS_PER_TILE = N // 16
RCHUNKS = ROWS_PER_TILE // 128
KROWS_PER_TILE = K // 16


def _zero_shared(s, zbuf, shared):
    def zfill(i, _):
        for j in range(8):
            zbuf[i, pl.ds(j * 16, 16)] = jnp.zeros((16,), jnp.float32)
        return 0

    lax.fori_loop(0, 8, zfill, 0)

    def zcopy(t, _):
        pltpu.sync_copy(zbuf, shared.at[pl.ds(s * KROWS_PER_TILE + t * 8, 8)])
        return 0

    lax.fori_loop(0, KROWS_PER_TILE // 8, zcopy, 0)


def _scatter_body(x_hbm, idx_hbm, esum_hbm, idx_v, rows_v, zbuf, accum):
    c = lax.axis_index("c")
    s = lax.axis_index("s")
    _zero_shared(s, zbuf, accum)
    plsc.subcore_barrier()

    base = s * ROWS_PER_TILE

    def chunk(j, _):
        pltpu.sync_copy(idx_hbm.at[pl.ds(base + j * 128, 128)], idx_v.at[j])
        pltpu.sync_copy(
            x_hbm.at[pl.ds(base + j * 128, 128), pl.ds(c * 128, 128)], rows_v)
        pltpu.sync_copy(rows_v, accum.at[idx_v.at[j]], add=True)
        return 0

    lax.fori_loop(0, RCHUNKS, chunk, 0)
    plsc.subcore_barrier()

    pltpu.sync_copy(
        accum.at[pl.ds(s * KROWS_PER_TILE, KROWS_PER_TILE)],
        esum_hbm.at[pl.ds(s * KROWS_PER_TILE, KROWS_PER_TILE),
                    pl.ds(c * 128, 128)])


def _counts_body(idx_hbm, cnt_hbm, idx_v, ones_v, zbuf, cacc):
    c = lax.axis_index("c")
    s = lax.axis_index("s")

    @pl.when(c == 0)
    def _():
        def ofill(i, _):
            for j in range(8):
                ones_v[i, pl.ds(j * 16, 16)] = jnp.ones((16,), jnp.float32)
            return 0

        lax.fori_loop(0, 128, ofill, 0)
        _zero_shared(s, zbuf, cacc)

    plsc.subcore_barrier()

    @pl.when(c == 0)
    def _():
        base = s * ROWS_PER_TILE

        def chunk(j, _):
            pltpu.sync_copy(idx_hbm.at[pl.ds(base + j * 128, 128)],
                            idx_v.at[j])
            pltpu.sync_copy(ones_v, cacc.at[idx_v.at[j]], add=True)
            return 0

        lax.fori_loop(0, RCHUNKS, chunk, 0)

    plsc.subcore_barrier()

    @pl.when(c == 0)
    def _():
        pltpu.sync_copy(
            cacc.at[pl.ds(s * KROWS_PER_TILE, KROWS_PER_TILE)],
            cnt_hbm.at[pl.ds(s * KROWS_PER_TILE, KROWS_PER_TILE)])


def _norm_body(cs_ref, cnt_ref, eavg_ref, esum_ref, out_ref):
    cn = cs_ref[...] * DECAY + cnt_ref[...] * (1.0 - DECAY)
    total = jnp.sum(cn)
    smoothed = (cn + EPS) / (total + K * EPS) * total
    eavg_new = eavg_ref[...] * DECAY + esum_ref[...] * (1.0 - DECAY)
    out_ref[...] = eavg_new / smoothed[:, None]


def _normalize(cs, cnt, eavg, esum):
    return pl.pallas_call(
        _norm_body,
        out_shape=jax.ShapeDtypeStruct((K, D), jnp.float32),
    )(cs, cnt, eavg, esum)


B_PER_W = N // 32
GCHUNK = 96


def _gather_body(table_hbm, idx_hbm, out_hbm, idx_v, rows_v, sem):
    c = lax.axis_index("c")
    s = lax.axis_index("s")
    wid = s * 2 + c
    base = wid * B_PER_W
    pltpu.sync_copy(idx_hbm.at[pl.ds(base, B_PER_W)], idx_v)

    def chunk(j, _):
        pltpu.async_copy(
            table_hbm.at[idx_v.at[pl.ds(j * GCHUNK, GCHUNK)]], rows_v, sem
        ).wait()
        pltpu.sync_copy(rows_v, out_hbm.at[pl.ds(base + j * GCHUNK, GCHUNK)])
        return 0

    lax.fori_loop(0, B_PER_W // GCHUNK, chunk, 0)


@functools.cache
def _sc_kernels():
    mesh = plsc.VectorSubcoreMesh(core_axis_name="c", subcore_axis_name="s")
    scatter = functools.partial(
        pl.kernel,
        mesh=mesh,
        out_type=jax.ShapeDtypeStruct((K, D), jnp.float32),
        scratch_types=[
            pltpu.VMEM((RCHUNKS, 128), jnp.int32),
            pltpu.VMEM((128, 128), jnp.float32),
            pltpu.VMEM((8, 128), jnp.float32),
            pltpu.VMEM_SHARED((K, 128), jnp.float32),
        ],
    )(_scatter_body)
    counts = functools.partial(
        pl.kernel,
        mesh=mesh,
        out_type=jax.ShapeDtypeStruct((K, 128), jnp.float32),
        scratch_types=[
            pltpu.VMEM((RCHUNKS, 128), jnp.int32),
            pltpu.VMEM((128, 128), jnp.float32),
            pltpu.VMEM((8, 128), jnp.float32),
            pltpu.VMEM_SHARED((K, 128), jnp.float32),
        ],
    )(_counts_body)
    gather = functools.partial(
        pl.kernel,
        mesh=mesh,
        out_type=jax.ShapeDtypeStruct((N, D), jnp.float32),
        scratch_types=[
            pltpu.VMEM((B_PER_W,), jnp.int32),
            pltpu.VMEM((GCHUNK, D), jnp.float32),
            pltpu.SemaphoreType.DMA,
        ],
    )(_gather_body)
    return scatter, counts, gather


def kernel(inputs_flatten, embed, cluster_size, embed_avg):
    scatter, counts, gather = _sc_kernels()
    idx = _argmin(inputs_flatten, embed, _e2(embed))
    esum = scatter(inputs_flatten, idx)
    cnt128 = counts(idx)
    enorm = _normalize(cluster_size, cnt128[:, 0], embed_avg, esum)
    quantize = gather(enorm, idx)
    return quantize, idx[:, None]
